# baseline (speedup 1.0000x reference)
import functools

import jax
import jax.numpy as jnp
from jax import lax
from jax.experimental import pallas as pl
from jax.experimental.pallas import tpu as pltpu
from jax.experimental.pallas import tpu_sc as plsc

B = 16
F = 6
M = 8
K = 10
T = 32768
BT = 4096
NSUB = 16
TW = T // NSUB


def _tc_dense_body(xT_ref, cal_ref, lat_ref, tok_ref):
    x = xT_ref[...]

    x9 = x * 9.0
    ws = [jnp.clip(x9 - float(k), 0.0, 1.0) for k in range(K - 1)]

    cal = cal_ref[...]
    lat = lat_ref[...]

    for m in range(M):
        cm = jnp.zeros((F, BT), jnp.float32) + cal[:, m * K:m * K + 1]
        for k in range(K - 1):
            cm = cm + ws[k] * cal[:, m * K + k + 1:m * K + k + 2]
        cm = jnp.clip(cm, 0.0, 1.0)
        latcol = lat[:, m:m + 1]
        half = (2 ** F) // 2
        x0 = cm[0:1, :]
        vals = latcol[:half] + (latcol[half:] - latcol[:half]) * x0
        for d in range(1, F):
            half //= 2
            xd = cm[d:d + 1, :]
            vals = vals[:half] + (vals[half:] - vals[:half]) * xd
        tok_ref[m:m + 1, :] = vals


def _run_tc_dense(xT, cal2, lat2):
    nblk = T // BT
    return pl.pallas_call(
        _tc_dense_body,
        grid=(nblk,),
        in_specs=[
            pl.BlockSpec((F, BT), lambda i: (0, i)),
            pl.BlockSpec((F, M * K), lambda i: (0, 0)),
            pl.BlockSpec((2 ** F, M), lambda i: (0, 0)),
        ],
        out_specs=pl.BlockSpec((M, BT), lambda i: (0, i)),
        out_shape=jax.ShapeDtypeStruct((M, T), jnp.float32),
    )(xT, cal2, lat2)


def _sc_agg_body(tok_hbm, seg_hbm, acc_hbm, rows_v, seg_v, stage_v, acc_sh):
    c = lax.axis_index("c")
    s = lax.axis_index("s")

    @pl.when(c == 0)
    def _():
        @pl.when(s == 0)
        def _():
            for i in range(B):
                stage_v[i] = jnp.zeros((16,), jnp.float32)
            pltpu.sync_copy(stage_v, acc_sh)

        plsc.subcore_barrier()

        base = s * TW
        pltpu.sync_copy(tok_hbm.at[pl.ds(base, TW)], rows_v)
        pltpu.sync_copy(seg_hbm.at[pl.ds(base, TW)], seg_v)
        pltpu.sync_copy(rows_v, acc_sh.at[seg_v], add=True)

        plsc.subcore_barrier()

        @pl.when(s == 0)
        def _():
            pltpu.sync_copy(acc_sh, acc_hbm)


def _make_sc_agg():
    mesh = plsc.VectorSubcoreMesh(core_axis_name="c", subcore_axis_name="s")
    return pl.kernel(
        _sc_agg_body,
        mesh=mesh,
        compiler_params=pltpu.CompilerParams(use_tc_tiling_on_sc=False),
        out_type=jax.ShapeDtypeStruct((B, 16), jnp.float32),
        scratch_types=[
            pltpu.VMEM((TW, 16), jnp.float32),
            pltpu.VMEM((TW,), jnp.int32),
            pltpu.VMEM((B, 16), jnp.float32),
            pltpu.VMEM_SHARED((B, 16), jnp.float32),
        ],
    )


def _tc_tail_body(acc_ref, midkT_ref, fin_ref, out_ref):
    acc = acc_ref[...]
    midkT = midkT_ref[...]
    agg = acc[:, :M] / jnp.maximum(acc[:, M:M + 1], 1.0)
    mid = jnp.zeros((B, M), jnp.float32) + midkT[0:1, :]
    for k in range(K - 1):
        kp = -1.0 + 2.0 * k / (K - 1)
        wmk = jnp.clip((agg - kp) * ((K - 1) / 2.0), 0.0, 1.0)
        mid = mid + wmk * midkT[k + 1:k + 2, :]
    mid = jnp.clip(mid, 0.0, 1.0)
    vals = jnp.zeros((B, 2 ** M), jnp.float32) + fin_ref[...]
    half = (2 ** M) // 2
    for d in range(M):
        vals = (vals[:, :half]
                + (vals[:, half:] - vals[:, :half]) * mid[:, d:d + 1])
        half //= 2
    out_ref[...] = vals


def _run_tc_tail(acc, midkT, fin2):
    return pl.pallas_call(
        _tc_tail_body,
        in_specs=[
            pl.BlockSpec((B, 16), lambda: (0, 0)),
            pl.BlockSpec((K, M), lambda: (0, 0)),
            pl.BlockSpec((1, 2 ** M), lambda: (0, 0)),
        ],
        out_specs=pl.BlockSpec((B, 1), lambda: (0, 0)),
        out_shape=jax.ShapeDtypeStruct((B, 1), jnp.float32),
    )(acc, midkT, fin2)


@jax.jit
def _run(flat, segment_ids, calib_kernel, lattice_kernel, mid_kernel,
         final_kernel):
    xT = flat.T
    cal2 = jnp.transpose(calib_kernel, (1, 0, 2)).reshape(F, M * K)
    lat2 = lattice_kernel.T
    tok_mt = _run_tc_dense(xT, cal2, lat2)
    tok_pad = jnp.concatenate(
        [tok_mt.T, jnp.ones((T, 1), jnp.float32),
         jnp.zeros((T, 16 - M - 1), jnp.float32)], axis=1)
    seg = segment_ids.astype(jnp.int32)
    acc = _make_sc_agg()(tok_pad, seg)
    return _run_tc_tail(acc, mid_kernel.T, final_kernel.reshape(1, 2 ** M))


def kernel(flat, segment_ids, calib_kernel, lattice_kernel, mid_kernel,
           final_kernel):
    return _run(flat, segment_ids, calib_kernel, lattice_kernel, mid_kernel,
                final_kernel)


# speedup vs baseline: 1.5740x; 1.5740x over previous
import functools

import jax
import jax.numpy as jnp
from jax.experimental import pallas as pl
from jax.experimental.pallas import tpu as pltpu

B = 16
F = 6
M = 8
K = 10
T = 32768
BT = 4096
NW = F * (K - 1)
NC = M * F


def _tc_body(x_ref, seg_ref, rmat_ref, koff_ref, dmat_ref, bias_ref,
             lat_ref, midk_ref, fin_ref, out_ref, acc_ref, cnt_ref):
    pid = pl.program_id(0)
    nblk = pl.num_programs(0)

    xb = x_ref[...]
    seg = seg_ref[...]

    iota_b = jax.lax.broadcasted_iota(jnp.int32, (BT, B), 1)
    onehot = (seg == iota_b).astype(jnp.float32)

    xr9 = jax.lax.dot_general(
        rmat_ref[...], xb, (((1,), (1,)), ((), ())),
        preferred_element_type=jnp.float32)
    w = jnp.clip(xr9 - koff_ref[...], 0.0, 1.0)
    calib = jnp.dot(dmat_ref[...], w, preferred_element_type=jnp.float32)
    calib = jnp.clip(calib + bias_ref[...], 0.0, 1.0)

    lat = lat_ref[...]
    touts = []
    for m in range(M):
        latcol = lat[:, m:m + 1]
        half = (2 ** F) // 2
        x0 = calib[m * F:m * F + 1, :]
        vals = latcol[:half] + (latcol[half:] - latcol[:half]) * x0
        for d in range(1, F):
            half //= 2
            xd = calib[m * F + d:m * F + d + 1, :]
            vals = vals[:half] + (vals[half:] - vals[:half]) * xd
        touts.append(vals)

    tok = jnp.concatenate(touts, axis=0)
    psum = jnp.dot(tok, onehot, preferred_element_type=jnp.float32)
    pcnt = jnp.sum(onehot, axis=0, keepdims=True)

    @pl.when(pid == 0)
    def _():
        acc_ref[...] = psum
        cnt_ref[...] = pcnt

    @pl.when(pid > 0)
    def _():
        acc_ref[...] += psum
        cnt_ref[...] += pcnt

    @pl.when(pid == nblk - 1)
    def _():
        agg = acc_ref[...] / jnp.maximum(cnt_ref[...], 1.0)
        midk = midk_ref[...]
        mid = jnp.zeros((M, B), jnp.float32) + midk[:, 0:1]
        for k in range(K - 1):
            kp = -1.0 + 2.0 * k / (K - 1)
            wmk = jnp.clip((agg - kp) * ((K - 1) / 2.0), 0.0, 1.0)
            mid = mid + wmk * midk[:, k + 1:k + 2]
        mid = jnp.clip(mid, 0.0, 1.0)
        fin = fin_ref[...]
        half = (2 ** M) // 2
        x0 = mid[0:1, :]
        vals = fin[:half] + (fin[half:] - fin[:half]) * x0
        for d in range(1, M):
            half //= 2
            xd = mid[d:d + 1, :]
            vals = vals[:half] + (vals[half:] - vals[:half]) * xd
        out_ref[...] = vals


@jax.jit
def _run(flat, segment_ids, calib_kernel, lattice_kernel, mid_kernel,
         final_kernel):
    seg2 = segment_ids.astype(jnp.int32).reshape(T, 1)
    frows = jnp.repeat(jnp.arange(F), K - 1)
    krows = jnp.tile(jnp.arange(K - 1), F)
    rmat = (9.0 * (jax.nn.one_hot(frows, F, dtype=jnp.float32)))
    koff = krows.astype(jnp.float32).reshape(NW, 1)
    deltas = calib_kernel[:, :, 1:]
    fmask = jax.nn.one_hot(frows, F, dtype=jnp.float32)
    dmat = jnp.einsum('mfk,wf,wk->mfw',
                      deltas,
                      fmask,
                      jax.nn.one_hot(krows, K - 1, dtype=jnp.float32)
                      ).reshape(NC, NW)
    bias = calib_kernel[:, :, 0].reshape(NC, 1)
    lat2 = lattice_kernel.T
    fin2 = final_kernel.reshape(2 ** M, 1)

    nblk = T // BT
    out = pl.pallas_call(
        _tc_body,
        grid=(nblk,),
        in_specs=[
            pl.BlockSpec((BT, F), lambda i: (i, 0)),
            pl.BlockSpec((BT, 1), lambda i: (i, 0)),
            pl.BlockSpec((NW, F), lambda i: (0, 0)),
            pl.BlockSpec((NW, 1), lambda i: (0, 0)),
            pl.BlockSpec((NC, NW), lambda i: (0, 0)),
            pl.BlockSpec((NC, 1), lambda i: (0, 0)),
            pl.BlockSpec((2 ** F, M), lambda i: (0, 0)),
            pl.BlockSpec((M, K), lambda i: (0, 0)),
            pl.BlockSpec((2 ** M, 1), lambda i: (0, 0)),
        ],
        out_specs=pl.BlockSpec((1, B), lambda i: (0, 0)),
        out_shape=jax.ShapeDtypeStruct((1, B), jnp.float32),
        scratch_shapes=[
            pltpu.VMEM((M, B), jnp.float32),
            pltpu.VMEM((1, B), jnp.float32),
        ],
    )(flat, seg2, rmat, koff, dmat, bias, lat2, mid_kernel, fin2)
    return out.reshape(B, 1)


def kernel(flat, segment_ids, calib_kernel, lattice_kernel, mid_kernel,
           final_kernel):
    return _run(flat, segment_ids, calib_kernel, lattice_kernel, mid_kernel,
                final_kernel)


# speedup vs baseline: 1.9301x; 1.2263x over previous
import functools

import jax
import jax.numpy as jnp
from jax.experimental import pallas as pl
from jax.experimental.pallas import tpu as pltpu

B = 16
F = 6
M = 8
K = 10
BT = 4096


def _tc_body(xT_ref, seg_ref, cal_ref, lat_ref, midk_ref, fin_ref,
             out_ref, acc_ref, cnt_ref):
    pid = pl.program_id(0)
    nblk = pl.num_programs(0)

    x = xT_ref[...]
    seg = seg_ref[...]

    iota_b = jax.lax.broadcasted_iota(jnp.int32, (BT, B), 1)
    onehot = (seg == iota_b).astype(jnp.float32)

    x9 = x * 9.0
    ws = [jnp.clip(x9 - float(k), 0.0, 1.0) for k in range(K - 1)]

    cal = cal_ref[...]
    lat = lat_ref[...]

    touts = []
    for m in range(M):
        cm = jnp.zeros((F, BT), jnp.float32) + cal[:, m * K:m * K + 1]
        for k in range(K - 1):
            cm = cm + ws[k] * cal[:, m * K + k + 1:m * K + k + 2]
        cm = jnp.clip(cm, 0.0, 1.0)
        latcol = lat[:, m:m + 1]
        half = (2 ** F) // 2
        x0 = cm[0:1, :]
        vals = latcol[:half] + (latcol[half:] - latcol[:half]) * x0
        for d in range(1, F):
            half //= 2
            xd = cm[d:d + 1, :]
            vals = vals[:half] + (vals[half:] - vals[:half]) * xd
        touts.append(vals)

    tok = jnp.concatenate(touts, axis=0)
    psum = jnp.dot(tok, onehot, preferred_element_type=jnp.float32)
    pcnt = jnp.sum(onehot, axis=0, keepdims=True)

    @pl.when(pid == 0)
    def _():
        acc_ref[...] = psum
        cnt_ref[...] = pcnt

    @pl.when(pid > 0)
    def _():
        acc_ref[...] += psum
        cnt_ref[...] += pcnt

    @pl.when(pid == nblk - 1)
    def _():
        agg = acc_ref[...] / jnp.maximum(cnt_ref[...], 1.0)
        midk = midk_ref[...]
        mid = jnp.zeros((M, B), jnp.float32) + midk[:, 0:1]
        for k in range(K - 1):
            kp = -1.0 + 2.0 * k / (K - 1)
            wmk = jnp.clip((agg - kp) * ((K - 1) / 2.0), 0.0, 1.0)
            mid = mid + wmk * midk[:, k + 1:k + 2]
        mid = jnp.clip(mid, 0.0, 1.0)
        fin = fin_ref[...]
        half = (2 ** M) // 2
        x0 = mid[0:1, :]
        vals = fin[:half] + (fin[half:] - fin[:half]) * x0
        for d in range(1, M):
            half //= 2
            xd = mid[d:d + 1, :]
            vals = vals[:half] + (vals[half:] - vals[:half]) * xd
        out_ref[...] = vals


@functools.partial(jax.jit, static_argnums=())
def _run_tc(xT, seg2, cal2, lat2, midk, fin2):
    T = xT.shape[1]
    nblk = T // BT
    grid = (nblk,)
    out = pl.pallas_call(
        _tc_body,
        grid=grid,
        in_specs=[
            pl.BlockSpec((F, BT), lambda i: (0, i)),
            pl.BlockSpec((BT, 1), lambda i: (i, 0)),
            pl.BlockSpec((F, M * K), lambda i: (0, 0)),
            pl.BlockSpec((2 ** F, M), lambda i: (0, 0)),
            pl.BlockSpec((M, K), lambda i: (0, 0)),
            pl.BlockSpec((2 ** M, 1), lambda i: (0, 0)),
        ],
        out_specs=pl.BlockSpec((1, B), lambda i: (0, 0)),
        out_shape=jax.ShapeDtypeStruct((1, B), jnp.float32),
        scratch_shapes=[
            pltpu.VMEM((M, B), jnp.float32),
            pltpu.VMEM((1, B), jnp.float32),
        ],
    )(xT, seg2, cal2, lat2, midk, fin2)
    return out


def kernel(flat, segment_ids, calib_kernel, lattice_kernel, mid_kernel,
           final_kernel):
    T = flat.shape[0]
    xT = flat.T
    seg2 = segment_ids.astype(jnp.int32).reshape(T, 1)
    cal2 = jnp.transpose(calib_kernel, (1, 0, 2)).reshape(F, M * K)
    lat2 = lattice_kernel.T
    fin2 = final_kernel.reshape(2 ** M, 1)
    out = _run_tc(xT, seg2, cal2, lat2, mid_kernel, fin2)
    return out.reshape(B, 1)


# speedup vs baseline: 3.1413x; 1.6275x over previous
import functools

import jax
import jax.numpy as jnp
from jax.experimental import pallas as pl
from jax.experimental.pallas import tpu as pltpu

B = 16
F = 6
M = 8
K = 10
BT = 4096


def _tc_body(xT_ref, seg_ref, cal_ref, lat_ref, midk_ref, fin_ref,
             out_ref, acc_ref):
    pid = pl.program_id(0)
    nblk = pl.num_programs(0)

    x = xT_ref[...]
    seg = seg_ref[...]

    iota_b = jax.lax.broadcasted_iota(jnp.int32, (B, BT), 0)
    onehot = (seg == iota_b).astype(jnp.float32)

    x9 = x * 9.0
    ws = [jnp.clip(x9 - float(k), 0.0, 1.0) for k in range(K - 1)]

    cal = cal_ref[...]
    lat = lat_ref[...]

    touts = []
    for m in range(M):
        cm = jnp.zeros((F, BT), jnp.float32) + cal[:, m * K:m * K + 1]
        for k in range(K - 1):
            cm = cm + ws[k] * cal[:, m * K + k + 1:m * K + k + 2]
        cm = jnp.clip(cm, 0.0, 1.0)
        latcol = lat[:, m:m + 1]
        half = (2 ** F) // 2
        x0 = cm[0:1, :]
        vals = latcol[:half] + (latcol[half:] - latcol[:half]) * x0
        for d in range(1, F):
            half //= 2
            xd = cm[d:d + 1, :]
            vals = vals[:half] + (vals[half:] - vals[:half]) * xd
        touts.append(vals)

    touts.append(jnp.ones((1, BT), jnp.float32))
    tok9 = jnp.concatenate(touts, axis=0)
    psum = jax.lax.dot_general(
        tok9, onehot, (((1,), (1,)), ((), ())),
        preferred_element_type=jnp.float32)

    @pl.when(pid == 0)
    def _():
        acc_ref[...] = psum

    @pl.when(pid > 0)
    def _():
        acc_ref[...] += psum

    @pl.when(pid == nblk - 1)
    def _():
        acc = acc_ref[...]
        agg = acc[:M] / jnp.maximum(acc[M:M + 1], 1.0)
        midk = midk_ref[...]
        mid = jnp.zeros((M, B), jnp.float32) + midk[:, 0:1]
        for k in range(K - 1):
            kp = -1.0 + 2.0 * k / (K - 1)
            wmk = jnp.clip((agg - kp) * ((K - 1) / 2.0), 0.0, 1.0)
            mid = mid + wmk * midk[:, k + 1:k + 2]
        mid = jnp.clip(mid, 0.0, 1.0)
        fin = fin_ref[...]
        half = (2 ** M) // 2
        x0 = mid[0:1, :]
        vals = fin[:half] + (fin[half:] - fin[:half]) * x0
        for d in range(1, M):
            half //= 2
            xd = mid[d:d + 1, :]
            vals = vals[:half] + (vals[half:] - vals[:half]) * xd
        out_ref[...] = vals


@jax.jit
def _run(flat, segment_ids, calib_kernel, lattice_kernel, mid_kernel,
         final_kernel):
    T = flat.shape[0]
    xT = flat.T
    segr = segment_ids.astype(jnp.int32).reshape(1, T)
    cal2 = jnp.transpose(calib_kernel, (1, 0, 2)).reshape(F, M * K)
    lat2 = lattice_kernel.T
    fin2 = final_kernel.reshape(2 ** M, 1)

    nblk = T // BT
    out = pl.pallas_call(
        _tc_body,
        grid=(nblk,),
        in_specs=[
            pl.BlockSpec((F, BT), lambda i: (0, i)),
            pl.BlockSpec((1, BT), lambda i: (0, i)),
            pl.BlockSpec((F, M * K), lambda i: (0, 0)),
            pl.BlockSpec((2 ** F, M), lambda i: (0, 0)),
            pl.BlockSpec((M, K), lambda i: (0, 0)),
            pl.BlockSpec((2 ** M, 1), lambda i: (0, 0)),
        ],
        out_specs=pl.BlockSpec((1, B), lambda i: (0, 0)),
        out_shape=jax.ShapeDtypeStruct((1, B), jnp.float32),
        scratch_shapes=[
            pltpu.VMEM((M + 1, B), jnp.float32),
        ],
    )(xT, segr, cal2, lat2, mid_kernel, fin2)
    return out.reshape(B, 1)


def kernel(flat, segment_ids, calib_kernel, lattice_kernel, mid_kernel,
           final_kernel):
    return _run(flat, segment_ids, calib_kernel, lattice_kernel, mid_kernel,
                final_kernel)


# speedup vs baseline: 4.8561x; 1.5459x over previous
import functools

import jax
import jax.numpy as jnp
from jax.experimental import pallas as pl
from jax.experimental.pallas import tpu as pltpu

B = 16
F = 6
M = 8
K = 10
BT = 4096
NW = F * (K - 1)


def _tc_body(xT_ref, seg_ref, rmat_ref, koff_ref, dmat_ref, bias_ref,
             lbig_ref, midk_ref, fin_ref, out_ref, acc_ref):
    pid = pl.program_id(0)
    nblk = pl.num_programs(0)

    x = xT_ref[...]
    seg = seg_ref[...]

    iota_b = jax.lax.broadcasted_iota(jnp.int32, (B, BT), 0)
    onehot = (seg == iota_b).astype(jnp.float32)

    xr9 = jnp.dot(rmat_ref[...], x, preferred_element_type=jnp.float32)
    w = jnp.clip(xr9 - koff_ref[...], 0.0, 1.0)
    calib = jnp.dot(dmat_ref[...], w, preferred_element_type=jnp.float32)
    calib = jnp.clip(calib + bias_ref[...], 0.0, 1.0)
    X = [calib[f * M:(f + 1) * M] for f in range(F)]

    a1 = jnp.concatenate([1.0 - X[3], X[3]], 0)
    p2 = jnp.concatenate([a1[:M] * X[4], a1[M:] * X[4]], 0)
    a2 = jnp.concatenate([a1 - p2, p2], 0)
    p3 = jnp.concatenate([a2[i * M:(i + 1) * M] * X[5]
                          for i in range(4)], 0)
    a3 = jnp.concatenate([a2 - p3, p3], 0)

    V = jnp.dot(lbig_ref[...], a3, preferred_element_type=jnp.float32)

    d1 = V[32:] - V[:32]
    e1 = jnp.concatenate([d1[i * M:(i + 1) * M] * X[0]
                          for i in range(4)], 0)
    v32 = V[:32] + e1
    d2 = v32[16:] - v32[:16]
    e2 = jnp.concatenate([d2[:M] * X[1], d2[M:] * X[1]], 0)
    v16 = v32[:16] + e2
    d3 = v16[M:] - v16[:M]
    tok = v16[:M] + d3 * X[2]

    tok9 = jnp.concatenate([tok, jnp.ones((1, BT), jnp.float32)], 0)
    psum = jax.lax.dot_general(
        tok9, onehot, (((1,), (1,)), ((), ())),
        preferred_element_type=jnp.float32)

    @pl.when(pid == 0)
    def _():
        acc_ref[...] = psum

    @pl.when(pid > 0)
    def _():
        acc_ref[...] += psum

    @pl.when(pid == nblk - 1)
    def _():
        acc = acc_ref[...]
        agg = acc[:M] / jnp.maximum(acc[M:M + 1], 1.0)
        midk = midk_ref[...]
        mid = jnp.zeros((M, B), jnp.float32) + midk[:, 0:1]
        for k in range(K - 1):
            kp = -1.0 + 2.0 * k / (K - 1)
            wmk = jnp.clip((agg - kp) * ((K - 1) / 2.0), 0.0, 1.0)
            mid = mid + wmk * midk[:, k + 1:k + 2]
        mid = jnp.clip(mid, 0.0, 1.0)
        fin = fin_ref[...]
        half = (2 ** M) // 2
        x0 = mid[0:1, :]
        vals = fin[:half] + (fin[half:] - fin[:half]) * x0
        for d in range(1, M):
            half //= 2
            xd = mid[d:d + 1, :]
            vals = vals[:half] + (vals[half:] - vals[:half]) * xd
        out_ref[...] = vals


@jax.jit
def _run(flat, segment_ids, calib_kernel, lattice_kernel, mid_kernel,
         final_kernel):
    T = flat.shape[0]
    xT = flat.T
    segr = segment_ids.astype(jnp.int32).reshape(1, T)

    frows = jnp.repeat(jnp.arange(F), K - 1)
    krows = jnp.tile(jnp.arange(K - 1), F)
    rmat = 9.0 * jax.nn.one_hot(frows, F, dtype=jnp.float32)
    koff = krows.astype(jnp.float32).reshape(NW, 1)
    deltas = calib_kernel[:, :, 1:]
    dmat = jnp.einsum('mfk,wf,wk->fmw',
                      deltas,
                      jax.nn.one_hot(frows, F, dtype=jnp.float32),
                      jax.nn.one_hot(krows, K - 1, dtype=jnp.float32)
                      ).reshape(F * M, NW)
    bias = calib_kernel[:, :, 0].T.reshape(F * M, 1)
    rev = jnp.array([0, 4, 2, 6, 1, 5, 3, 7])
    l3d = lattice_kernel.reshape(M, 8, 8)[:, :, rev]
    lbig = jnp.einsum('mpq,mn->pmqn', l3d,
                      jnp.eye(M, dtype=jnp.float32)).reshape(64, 64)
    fin2 = final_kernel.reshape(2 ** M, 1)

    nblk = T // BT
    out = pl.pallas_call(
        _tc_body,
        grid=(nblk,),
        in_specs=[
            pl.BlockSpec((F, BT), lambda i: (0, i)),
            pl.BlockSpec((1, BT), lambda i: (0, i)),
            pl.BlockSpec((NW, F), lambda i: (0, 0)),
            pl.BlockSpec((NW, 1), lambda i: (0, 0)),
            pl.BlockSpec((F * M, NW), lambda i: (0, 0)),
            pl.BlockSpec((F * M, 1), lambda i: (0, 0)),
            pl.BlockSpec((64, 64), lambda i: (0, 0)),
            pl.BlockSpec((M, K), lambda i: (0, 0)),
            pl.BlockSpec((2 ** M, 1), lambda i: (0, 0)),
        ],
        out_specs=pl.BlockSpec((1, B), lambda i: (0, 0)),
        out_shape=jax.ShapeDtypeStruct((1, B), jnp.float32),
        scratch_shapes=[
            pltpu.VMEM((M + 1, B), jnp.float32),
        ],
    )(xT, segr, rmat, koff, dmat, bias, lbig, mid_kernel, fin2)
    return out.reshape(B, 1)


def kernel(flat, segment_ids, calib_kernel, lattice_kernel, mid_kernel,
           final_kernel):
    return _run(flat, segment_ids, calib_kernel, lattice_kernel, mid_kernel,
                final_kernel)
